# baseline (speedup 1.0000x reference)
import jax
import jax.numpy as jnp
from jax.experimental import pallas as pl
from jax.experimental.pallas import tpu as pltpu

_D = 128
_N = 16384
_B = 32
_WIN = 9
_YW = 16
_LOG2E = 1.4426950408889634


def _splat_kernel(zvol_ref, sz_ref, sy_ref, cz_ref, cy_ref, cx_ref, s0_ref,
                  s1_ref, s2_ref, qw_ref, qx_ref, qy_ref, qz_ref, rre_ref,
                  rim_ref, out_ref, patch_ref, zsem):
    step = pl.program_id(0)

    @pl.when(step == 0)
    def _init():
        cp = pltpu.make_async_copy(zvol_ref, out_ref, zsem)
        cp.start()
        cp.wait()

    span = float(_D - 1)

    def rd(ref):
        return ref[...].reshape(_B, 1, 1)

    cz, cy, cx = rd(cz_ref) * span, rd(cy_ref) * span, rd(cx_ref) * span
    s0 = jnp.clip(rd(s0_ref), 0.001, None)
    s1 = jnp.clip(rd(s1_ref), 0.001, None)
    s2 = jnp.clip(rd(s2_ref), 0.001, None)
    qw, qx, qy, qz = rd(qw_ref), rd(qx_ref), rd(qy_ref), rd(qz_ref)
    rho_re, rho_im = rd(rre_ref), rd(rim_ref)

    qnorm = jnp.clip(jnp.sqrt(qw * qw + qx * qx + qy * qy + qz * qz),
                     1e-6, None)
    qw, qx, qy, qz = qw / qnorm, qx / qnorm, qy / qnorm, qz / qnorm
    ww, xx, yy, zz = qw * qw, qx * qx, qy * qy, qz * qz
    wx, wy, wz = qw * qx, qw * qy, qw * qz
    xy, xz, yz = qx * qy, qx * qz, qy * qz
    r00 = ww + xx - yy - zz
    r01 = 2.0 * (xy - wz)
    r02 = 2.0 * (xz + wy)
    r10 = 2.0 * (xy + wz)
    r11 = ww - xx + yy - zz
    r12 = 2.0 * (yz - wx)
    r20 = 2.0 * (xz - wy)
    r21 = 2.0 * (yz + wx)
    r22 = ww - xx - yy + zz
    i0d = 1.0 / (s0 * s0)
    i1d = 1.0 / (s1 * s1)
    i2d = 1.0 / (s2 * s2)
    p00 = r00 * r00 * i0d + r01 * r01 * i1d + r02 * r02 * i2d
    p11 = r10 * r10 * i0d + r11 * r11 * i1d + r12 * r12 * i2d
    p22 = r20 * r20 * i0d + r21 * r21 * i1d + r22 * r22 * i2d
    p01 = r00 * r10 * i0d + r01 * r11 * i1d + r02 * r12 * i2d
    p02 = r00 * r20 * i0d + r01 * r21 * i1d + r02 * r22 * i2d
    p12 = r10 * r20 * i0d + r11 * r21 * i1d + r12 * r22 * i2d

    rad = jnp.maximum(jnp.maximum(s0, s1), s2) * 3.0
    bz = jnp.floor(cz) - float(_WIN // 2)
    by = jnp.floor(cy) - float(_WIN // 2)
    bx = jnp.floor(cx) - float(_WIN // 2)
    szf = jnp.clip(bz, 0.0, float(_D - _WIN))
    syf = jnp.floor(jnp.clip(by, 0.0, float(_D - _WIN)) / 8.0) * 8.0
    loz = jnp.maximum(0.0, jnp.floor(cz - rad))
    hiz = jnp.minimum(span, jnp.ceil(cz + rad))
    loy = jnp.maximum(0.0, jnp.floor(cy - rad))
    hiy = jnp.minimum(span, jnp.ceil(cy + rad))
    lox = jnp.maximum(0.0, jnp.floor(cx - rad))
    hix = jnp.minimum(span, jnp.ceil(cx + rad))

    pmask = (rho_re * rho_re + rho_im * rho_im >= 1e-12).astype(jnp.float32)

    ky = jax.lax.broadcasted_iota(jnp.int32, (1, _YW, 1), 1).astype(
        jnp.float32)
    ycoord = syf + ky
    xcoord = jax.lax.broadcasted_iota(jnp.int32, (1, 1, _D), 2).astype(
        jnp.float32)
    my = ((ycoord >= by) & (ycoord <= by + 8.0)
          & (ycoord >= loy) & (ycoord <= hiy)).astype(jnp.float32)
    mx = ((xcoord >= bx) & (xcoord <= bx + 8.0)
          & (xcoord >= lox) & (xcoord <= hix)).astype(jnp.float32)
    myx = my * mx * pmask

    ry = ycoord - cy
    rx = xcoord - cx
    tyy = p11 * ry * ry
    txx = p22 * rx * rx
    tyx = tyy + txx + 2.0 * p12 * ry * rx
    cy_lin = 2.0 * (p01 * ry + p02 * rx)

    for j in range(_WIN):
        zc = szf + float(j)
        mz = ((zc >= bz) & (zc <= bz + 8.0)
              & (zc >= loz) & (zc <= hiz)).astype(jnp.float32)
        rz = zc - cz
        e = tyx + rz * (p00 * rz + cy_lin)
        w = jnp.exp2(jnp.maximum(e * (-0.5 * _LOG2E), -120.0)) * (myx * mz)
        w4 = w.reshape(_B, _YW // 8, 8, _D)
        patch_ref[0, :, j] = w4 * rho_re.reshape(_B, 1, 1, 1)
        patch_ref[1, :, j] = w4 * rho_im.reshape(_B, 1, 1, 1)

    def body(b, carry):
        zs = sz_ref[0, 0, b]
        yg = sy_ref[0, 0, b]
        out_ref[:, pl.ds(zs, _WIN), pl.ds(yg, _YW // 8)] += patch_ref[:, b]
        return carry

    jax.lax.fori_loop(0, _B, body, 0)


@jax.jit
def kernel(centers, scales, quats, rho):
    span = float(_D - 1)
    cvz = centers[:, 0] * span
    cvy = centers[:, 1] * span
    sz = jnp.clip(jnp.floor(cvz).astype(jnp.int32) - _WIN // 2, 0,
                  _D - _WIN).reshape(_N // _B, 1, _B)
    sy = (jnp.clip(jnp.floor(cvy).astype(jnp.int32) - _WIN // 2, 0,
                   _D - _WIN) // 8).reshape(_N // _B, 1, _B)

    cols = (centers[:, 0:1], centers[:, 1:2], centers[:, 2:3],
            scales[:, 0:1], scales[:, 1:2], scales[:, 2:3],
            quats[:, 0:1], quats[:, 1:2], quats[:, 2:3], quats[:, 3:4],
            rho[:, 0:1], rho[:, 1:2])

    smem_spec = pl.BlockSpec((1, 1, _B), lambda i: (i, 0, 0),
                             memory_space=pltpu.SMEM)
    col_spec = pl.BlockSpec((_B, 1), lambda i: (i, 0))
    zvol = jnp.zeros((2, _D, _D // 8, 8, _D), jnp.float32)
    return pl.pallas_call(
        _splat_kernel,
        grid=(_N // _B,),
        in_specs=([pl.BlockSpec(memory_space=pl.ANY),
                   smem_spec, smem_spec] + [col_spec] * 12),
        out_specs=pl.BlockSpec((2, _D, _D // 8, 8, _D),
                               lambda i: (0, 0, 0, 0, 0)),
        scratch_shapes=[
            pltpu.VMEM((2, _B, _WIN, _YW // 8, 8, _D), jnp.float32),
            pltpu.SemaphoreType.DMA],
        out_shape=jax.ShapeDtypeStruct((2, _D, _D // 8, 8, _D), jnp.float32),
    )(zvol, sz, sy, *cols).reshape(2, _D, _D, _D)


# speedup vs baseline: 1.1294x; 1.1294x over previous
import jax
import jax.numpy as jnp
from jax.experimental import pallas as pl
from jax.experimental.pallas import tpu as pltpu

_D = 128
_N = 16384
_B = 32
_WIN = 9
_YW = 16
_LOG2E = 1.4426950408889634


def _splat_kernel(zvol_ref, sz_ref, sy_ref, cz_ref, cy_ref, cx_ref, s0_ref,
                  s1_ref, s2_ref, qw_ref, qx_ref, qy_ref, qz_ref, rre_ref,
                  rim_ref, out_ref, patch_ref, zsem):
    step = pl.program_id(0)

    @pl.when(step == 0)
    def _init():
        cp = pltpu.make_async_copy(zvol_ref, out_ref, zsem)
        cp.start()
        cp.wait()

    span = float(_D - 1)

    def rd(ref):
        return ref[...].reshape(_B, 1, 1)

    cz, cy, cx = rd(cz_ref) * span, rd(cy_ref) * span, rd(cx_ref) * span
    s0 = jnp.clip(rd(s0_ref), 0.001, None)
    s1 = jnp.clip(rd(s1_ref), 0.001, None)
    s2 = jnp.clip(rd(s2_ref), 0.001, None)
    qw, qx, qy, qz = rd(qw_ref), rd(qx_ref), rd(qy_ref), rd(qz_ref)
    rho_re, rho_im = rd(rre_ref), rd(rim_ref)

    qnorm = jnp.clip(jnp.sqrt(qw * qw + qx * qx + qy * qy + qz * qz),
                     1e-6, None)
    qw, qx, qy, qz = qw / qnorm, qx / qnorm, qy / qnorm, qz / qnorm
    ww, xx, yy, zz = qw * qw, qx * qx, qy * qy, qz * qz
    wx, wy, wz = qw * qx, qw * qy, qw * qz
    xy, xz, yz = qx * qy, qx * qz, qy * qz
    r00 = ww + xx - yy - zz
    r01 = 2.0 * (xy - wz)
    r02 = 2.0 * (xz + wy)
    r10 = 2.0 * (xy + wz)
    r11 = ww - xx + yy - zz
    r12 = 2.0 * (yz - wx)
    r20 = 2.0 * (xz - wy)
    r21 = 2.0 * (yz + wx)
    r22 = ww - xx - yy + zz
    i0d = 1.0 / (s0 * s0)
    i1d = 1.0 / (s1 * s1)
    i2d = 1.0 / (s2 * s2)
    p00 = r00 * r00 * i0d + r01 * r01 * i1d + r02 * r02 * i2d
    p11 = r10 * r10 * i0d + r11 * r11 * i1d + r12 * r12 * i2d
    p22 = r20 * r20 * i0d + r21 * r21 * i1d + r22 * r22 * i2d
    p01 = r00 * r10 * i0d + r01 * r11 * i1d + r02 * r12 * i2d
    p02 = r00 * r20 * i0d + r01 * r21 * i1d + r02 * r22 * i2d
    p12 = r10 * r20 * i0d + r11 * r21 * i1d + r12 * r22 * i2d

    rad = jnp.maximum(jnp.maximum(s0, s1), s2) * 3.0
    bz = jnp.floor(cz) - float(_WIN // 2)
    by = jnp.floor(cy) - float(_WIN // 2)
    bx = jnp.floor(cx) - float(_WIN // 2)
    szf = jnp.clip(bz, 0.0, float(_D - _WIN))
    syf = jnp.floor(jnp.clip(by, 0.0, float(_D - _WIN)) / 8.0) * 8.0
    loz = jnp.maximum(0.0, jnp.floor(cz - rad))
    hiz = jnp.minimum(span, jnp.ceil(cz + rad))
    loy = jnp.maximum(0.0, jnp.floor(cy - rad))
    hiy = jnp.minimum(span, jnp.ceil(cy + rad))
    lox = jnp.maximum(0.0, jnp.floor(cx - rad))
    hix = jnp.minimum(span, jnp.ceil(cx + rad))

    pmask = (rho_re * rho_re + rho_im * rho_im >= 1e-12).astype(jnp.float32)

    ky = jax.lax.broadcasted_iota(jnp.int32, (1, _YW, 1), 1).astype(
        jnp.float32)
    ycoord = syf + ky
    xcoord = jax.lax.broadcasted_iota(jnp.int32, (1, 1, _D), 2).astype(
        jnp.float32)
    my = ((ycoord >= by) & (ycoord <= by + 8.0)
          & (ycoord >= loy) & (ycoord <= hiy)).astype(jnp.float32)
    mx = ((xcoord >= bx) & (xcoord <= bx + 8.0)
          & (xcoord >= lox) & (xcoord <= hix)).astype(jnp.float32)
    myx = my * mx * pmask

    ry = ycoord - cy
    rx = xcoord - cx
    tyy = p11 * ry * ry
    txx = p22 * rx * rx
    tyx = tyy + txx + 2.0 * p12 * ry * rx
    cy_lin = 2.0 * (p01 * ry + p02 * rx)

    for j in range(_WIN):
        zc = szf + float(j)
        mz = ((zc >= bz) & (zc <= bz + 8.0)
              & (zc >= loz) & (zc <= hiz)).astype(jnp.float32)
        rz = zc - cz
        e = tyx + rz * (p00 * rz + cy_lin)
        w = jnp.exp2(jnp.maximum(e * (-0.5 * _LOG2E), -120.0)) * (myx * mz)
        w4 = w.reshape(_B, _YW // 8, 8, _D)
        patch_ref[0, :, j] = w4 * rho_re.reshape(_B, 1, 1, 1)
        patch_ref[1, :, j] = w4 * rho_im.reshape(_B, 1, 1, 1)

    def body(b, carry):
        zs = sz_ref[0, 0, b]
        yg = sy_ref[0, 0, b]
        out_ref[:, pl.ds(zs, _WIN), pl.ds(yg, _YW // 8)] += patch_ref[:, b]
        return carry

    jax.lax.fori_loop(0, _B, body, 0, unroll=True)


@jax.jit
def kernel(centers, scales, quats, rho):
    span = float(_D - 1)
    cvz = centers[:, 0] * span
    cvy = centers[:, 1] * span
    sz = jnp.clip(jnp.floor(cvz).astype(jnp.int32) - _WIN // 2, 0,
                  _D - _WIN).reshape(_N // _B, 1, _B)
    sy = (jnp.clip(jnp.floor(cvy).astype(jnp.int32) - _WIN // 2, 0,
                   _D - _WIN) // 8).reshape(_N // _B, 1, _B)

    cols = (centers[:, 0:1], centers[:, 1:2], centers[:, 2:3],
            scales[:, 0:1], scales[:, 1:2], scales[:, 2:3],
            quats[:, 0:1], quats[:, 1:2], quats[:, 2:3], quats[:, 3:4],
            rho[:, 0:1], rho[:, 1:2])

    smem_spec = pl.BlockSpec((1, 1, _B), lambda i: (i, 0, 0),
                             memory_space=pltpu.SMEM)
    col_spec = pl.BlockSpec((_B, 1), lambda i: (i, 0))
    zvol = jnp.zeros((2, _D, _D // 8, 8, _D), jnp.float32)
    return pl.pallas_call(
        _splat_kernel,
        grid=(_N // _B,),
        in_specs=([pl.BlockSpec(memory_space=pl.ANY),
                   smem_spec, smem_spec] + [col_spec] * 12),
        out_specs=pl.BlockSpec((2, _D, _D // 8, 8, _D),
                               lambda i: (0, 0, 0, 0, 0)),
        scratch_shapes=[
            pltpu.VMEM((2, _B, _WIN, _YW // 8, 8, _D), jnp.float32),
            pltpu.SemaphoreType.DMA],
        out_shape=jax.ShapeDtypeStruct((2, _D, _D // 8, 8, _D), jnp.float32),
    )(zvol, sz, sy, *cols).reshape(2, _D, _D, _D)
